# baseline (speedup 1.0000x reference)
import functools

import jax
import jax.numpy as jnp
from jax import lax
from jax.experimental import pallas as pl
from jax.experimental.pallas import tpu as pltpu
from jax.experimental.pallas import tpu_sc as plsc

DIM = 64
N_TOK = 819200
NW = 32
BW = N_TOK // NW
CHUNK = 128
NCHUNK = BW // CHUNK


def _sc_pool(x2d, table):
    mesh = plsc.VectorSubcoreMesh(core_axis_name="c", subcore_axis_name="s")

    @functools.partial(
        pl.kernel,
        mesh=mesh,
        out_type=jax.ShapeDtypeStruct((NW, DIM), jnp.float32),
        scratch_types=[
            pltpu.VMEM((NCHUNK, CHUNK), jnp.int32),
            pltpu.VMEM((CHUNK, DIM), jnp.float32),
            pltpu.VMEM((1, DIM), jnp.float32),
            pltpu.SemaphoreType.DMA,
        ],
        compiler_params=pltpu.CompilerParams(use_tc_tiling_on_sc=False),
    )
    def body(x_hbm, v_hbm, out_hbm, idx_v, rows_v, acc_v, sem):
        wid = lax.axis_index("s") * 2 + lax.axis_index("c")
        pltpu.sync_copy(x_hbm.at[pl.ds(wid * NCHUNK, NCHUNK)], idx_v)

        zero = jnp.zeros((16,), jnp.float32)

        def chunk_body(g, accs):
            pltpu.async_copy(v_hbm.at[idx_v.at[g]], rows_v, sem).wait()

            def row_body(r, a):
                return tuple(
                    a[q] + rows_v[r, pl.ds(q * 16, 16)] for q in range(4)
                )

            return lax.fori_loop(0, CHUNK, row_body, accs)

        accs = lax.fori_loop(0, NCHUNK, chunk_body, (zero, zero, zero, zero))
        for q in range(4):
            acc_v[0, pl.ds(q * 16, 16)] = accs[q]
        pltpu.sync_copy(acc_v, out_hbm.at[pl.ds(wid, 1)])

    return body(x2d, table)


def _tc_head(partials, w, b2d):
    def body(p_ref, w_ref, b_ref, o_ref):
        v = jnp.sum(p_ref[...], axis=0, keepdims=True)
        h = jnp.maximum(v, 0.0)
        logits = lax.dot_general(
            h, w_ref[...], (((1,), (1,)), ((), ())),
            preferred_element_type=jnp.float32,
        ) + b_ref[...]
        m = jnp.max(logits, axis=1, keepdims=True)
        lse = m + jnp.log(jnp.sum(jnp.exp(logits - m), axis=1, keepdims=True))
        o_ref[...] = logits - lse

    return pl.pallas_call(
        body,
        out_shape=jax.ShapeDtypeStruct((1, 2), jnp.float32),
    )(partials, w, b2d)


def kernel(X, V, W, b):
    x2d = X.reshape(NW * NCHUNK, CHUNK)
    partials = _sc_pool(x2d, V)
    out = _tc_head(partials, W, b.reshape(1, 2))
    return out.reshape(2)


# speedup vs baseline: 1.3036x; 1.3036x over previous
import functools

import jax
import jax.numpy as jnp
from jax import lax
from jax.experimental import pallas as pl
from jax.experimental.pallas import tpu as pltpu
from jax.experimental.pallas import tpu_sc as plsc

DIM = 64
N_TOK = 819200
NW = 32
BW = N_TOK // NW
CHUNK = 128
NCHUNK = BW // CHUNK


NBUF = 8
UNROLL = 8


def _sc_pool(x2d, table):
    mesh = plsc.VectorSubcoreMesh(core_axis_name="c", subcore_axis_name="s")

    @functools.partial(
        pl.kernel,
        mesh=mesh,
        out_type=jax.ShapeDtypeStruct((NW, DIM), jnp.float32),
        scratch_types=[
            pltpu.VMEM((NCHUNK, CHUNK), jnp.int32),
            pltpu.VMEM((NBUF, CHUNK, DIM), jnp.float32),
            pltpu.VMEM((1, DIM), jnp.float32),
        ]
        + [pltpu.SemaphoreType.DMA] * NBUF,
        compiler_params=pltpu.CompilerParams(use_tc_tiling_on_sc=False),
    )
    def body(x_hbm, v_hbm, out_hbm, idx_v, rows_v, acc_v, *sems):
        wid = lax.axis_index("s") * 2 + lax.axis_index("c")
        pltpu.sync_copy(x_hbm.at[pl.ds(wid * NCHUNK, NCHUNK)], idx_v)

        def start(g, b):
            pltpu.async_copy(v_hbm.at[idx_v.at[g]], rows_v.at[b], sems[b])

        def wait(b):
            pltpu.make_async_copy(
                v_hbm.at[idx_v.at[0]], rows_v.at[b], sems[b]
            ).wait()

        def accum(b, accs):
            def rows8(i, a):
                r0 = i * UNROLL
                for j in range(UNROLL):
                    a = tuple(
                        a[q] + rows_v[b, r0 + j, pl.ds(q * 16, 16)]
                        for q in range(4)
                    )
                return a

            return lax.fori_loop(0, CHUNK // UNROLL, rows8, accs)

        for b in range(NBUF):
            start(b, b)

        zero = jnp.zeros((16,), jnp.float32)
        accs = (zero, zero, zero, zero)

        steady = NCHUNK // NBUF - 1

        def round_body(gg, a):
            for b in range(NBUF):
                g = gg * NBUF + b
                wait(b)
                a = accum(b, a)
                start(g + NBUF, b)
            return a

        accs = lax.fori_loop(0, steady, round_body, accs)
        for b in range(NBUF):
            wait(b)
            accs = accum(b, accs)

        for q in range(4):
            acc_v[0, pl.ds(q * 16, 16)] = accs[q]
        pltpu.sync_copy(acc_v, out_hbm.at[pl.ds(wid, 1)])

    return body(x2d, table)


def _tc_head(partials, w, b2d):
    def body(p_ref, w_ref, b_ref, o_ref):
        v = jnp.sum(p_ref[...], axis=0, keepdims=True)
        h = jnp.maximum(v, 0.0)
        logits = lax.dot_general(
            h, w_ref[...], (((1,), (1,)), ((), ())),
            preferred_element_type=jnp.float32,
        ) + b_ref[...]
        m = jnp.max(logits, axis=1, keepdims=True)
        lse = m + jnp.log(jnp.sum(jnp.exp(logits - m), axis=1, keepdims=True))
        o_ref[...] = logits - lse

    return pl.pallas_call(
        body,
        out_shape=jax.ShapeDtypeStruct((1, 2), jnp.float32),
    )(partials, w, b2d)


def kernel(X, V, W, b):
    x2d = X.reshape(NW * NCHUNK, CHUNK)
    partials = _sc_pool(x2d, V)
    out = _tc_head(partials, W, b.reshape(1, 2))
    return out.reshape(2)
